# baseline (speedup 1.0000x reference)
import functools

import jax
import jax.numpy as jnp
from jax import lax
from jax.experimental import pallas as pl
from jax.experimental.pallas import tpu as pltpu
from jax.experimental.pallas import tpu_sc as plsc

_NC = 2
_NS = 16
_NW = _NC * _NS
_D = 64
_VB = 8192


def _project_table(tt, w128, b128):
    v = tt.shape[1]

    def proj(t_ref, w_ref, b_ref, o_ref):
        r = lax.dot_general(
            t_ref[...], w_ref[...], (((0,), (0,)), ((), ())),
            preferred_element_type=jnp.float32)
        o_ref[...] = r + b_ref[...]

    return pl.pallas_call(
        proj,
        grid=(pl.cdiv(v, _VB),),
        in_specs=[
            pl.BlockSpec((_D, _VB), lambda i: (0, i)),
            pl.BlockSpec((_D, 2 * _D), lambda i: (0, 0)),
            pl.BlockSpec((1, 2 * _D), lambda i: (0, 0)),
        ],
        out_specs=pl.BlockSpec((_VB, 2 * _D), lambda i: (i, 0)),
        out_shape=jax.ShapeDtypeStruct((v, 2 * _D), jnp.float32),
    )(tt, w128, b128)


def _gather_out(xt, ptbl, bsz, seq):
    chunk = 128
    mesh = plsc.VectorSubcoreMesh(core_axis_name="c", subcore_axis_name="s")

    @functools.partial(
        pl.kernel,
        mesh=mesh,
        compiler_params=pltpu.CompilerParams(needs_layout_passes=False),
        out_type=jax.ShapeDtypeStruct((seq, _D, bsz), jnp.float32),
        scratch_types=[
            pltpu.VMEM((seq, chunk), jnp.int32),
            pltpu.VMEM((chunk, 2 * _D), jnp.float32),
            pltpu.VMEM((chunk, 2 * _D), jnp.float32),
            pltpu.VMEM((_D, chunk), jnp.float32),
            pltpu.VMEM((_D, chunk), jnp.float32),
            pltpu.SemaphoreType.DMA,
            pltpu.SemaphoreType.DMA,
            pltpu.SemaphoreType.DMA,
            pltpu.SemaphoreType.DMA,
        ],
    )
    def k(x_hbm, tbl_hbm, out_hbm, idx_v, rows0, rows1, st0, st1,
          sem0, sem1, osem0, osem1):
        wid = lax.axis_index("s") * _NC + lax.axis_index("c")
        b0 = wid * chunk
        pltpu.sync_copy(x_hbm.at[:, pl.ds(b0, chunk)], idx_v)

        rows = (rows0, rows1)
        stg = (st0, st1)
        sems = (sem0, sem1)
        osems = (osem0, osem1)

        def start(g, p):
            pltpu.async_copy(tbl_hbm.at[idx_v.at[g]], rows[p], sems[p])

        def work(g, p):
            pltpu.make_async_copy(
                tbl_hbm.at[idx_v.at[g]], rows[p], sems[p]).wait()
            lane_iota = lax.iota(jnp.int32, 16)

            def dt_body(d, carry):
                def jg_body(jg, c2):
                    vals = plsc.load_gather(
                        rows[p], [jg * 16 + lane_iota,
                                  jnp.full((16,), d, jnp.int32)])
                    stg[p][d, pl.ds(jg * 16, 16)] = vals
                    return c2

                lax.fori_loop(0, chunk // 16, jg_body, 0, unroll=8)
                return carry

            lax.fori_loop(0, _D, dt_body, 0, unroll=8)
            pltpu.async_copy(
                stg[p], out_hbm.at[g, :, pl.ds(b0, chunk)], osems[p])

        start(0, 0)

        def step(g, carry):
            p = lax.rem(g, 2)

            @pl.when(g + 1 < seq)
            def _():
                @pl.when(p == 0)
                def _():
                    start(g + 1, 1)

                @pl.when(p == 1)
                def _():
                    start(g + 1, 0)

            @pl.when(p == 0)
            def _():
                @pl.when(g >= 2)
                def _():
                    pltpu.make_async_copy(
                        st0, out_hbm.at[g - 2, :, pl.ds(b0, chunk)],
                        osem0).wait()
                work(g, 0)

            @pl.when(p == 1)
            def _():
                @pl.when(g >= 2)
                def _():
                    pltpu.make_async_copy(
                        st1, out_hbm.at[g - 2, :, pl.ds(b0, chunk)],
                        osem1).wait()
                work(g, 1)

            return carry

        lax.fori_loop(0, seq, step, 0)

        pltpu.make_async_copy(
            st0, out_hbm.at[seq - 2, :, pl.ds(b0, chunk)], osem0).wait()
        pltpu.make_async_copy(
            st1, out_hbm.at[seq - 1, :, pl.ds(b0, chunk)], osem1).wait()

    return k(xt, ptbl)


def kernel(x, table, W, b):
    bsz, seq = x.shape
    xt = x.astype(jnp.int32).T
    tt = table.T
    wt = W.T
    w128 = jnp.concatenate([wt, wt], axis=1)
    b128 = jnp.concatenate([b, b]).reshape(1, 2 * _D)
    ptbl = _project_table(tt, w128, b128)
    o3 = _gather_out(xt, ptbl, bsz, seq)
    return jnp.transpose(o3, (2, 0, 1))


# speedup vs baseline: 1.1258x; 1.1258x over previous
import functools

import jax
import jax.numpy as jnp
from jax import lax
from jax.experimental import pallas as pl
from jax.experimental.pallas import tpu as pltpu
from jax.experimental.pallas import tpu_sc as plsc

_NC = 2
_NS = 16
_NW = _NC * _NS
_D = 64
_VB = 8192


def _project_table(tt, w128, b128):
    v = tt.shape[1]

    def proj(t_ref, w_ref, b_ref, o_ref):
        r = lax.dot_general(
            t_ref[...], w_ref[...], (((0,), (0,)), ((), ())),
            preferred_element_type=jnp.float32)
        o_ref[...] = r + b_ref[...]

    return pl.pallas_call(
        proj,
        grid=(pl.cdiv(v, _VB),),
        in_specs=[
            pl.BlockSpec((_D, _VB), lambda i: (0, i)),
            pl.BlockSpec((_D, 2 * _D), lambda i: (0, 0)),
            pl.BlockSpec((1, 2 * _D), lambda i: (0, 0)),
        ],
        out_specs=pl.BlockSpec((_VB, 2 * _D), lambda i: (i, 0)),
        out_shape=jax.ShapeDtypeStruct((v, 2 * _D), jnp.float32),
    )(tt, w128, b128)


def _gather_out(x3, ptbl, bsz, seq):
    _, nchunk, chunk = x3.shape
    bpw = bsz // _NW
    mesh = plsc.VectorSubcoreMesh(core_axis_name="c", subcore_axis_name="s")

    @functools.partial(
        pl.kernel,
        mesh=mesh,
        out_type=jax.ShapeDtypeStruct((bsz, seq, 2 * _D), jnp.float32),
        scratch_types=[
            pltpu.VMEM((nchunk, chunk), jnp.int32),
            pltpu.VMEM((chunk, 2 * _D), jnp.float32),
            pltpu.VMEM((chunk, 2 * _D), jnp.float32),
            pltpu.SemaphoreType.DMA,
            pltpu.SemaphoreType.DMA,
        ],
    )
    def k(x_hbm, tbl_hbm, out_hbm, idx_v, rows0, rows1, sem0, sem1):
        wid = lax.axis_index("s") * _NC + lax.axis_index("c")
        b0 = wid * bpw
        pltpu.sync_copy(x_hbm.at[wid], idx_v)

        def start(g, buf, sem):
            pltpu.async_copy(tbl_hbm.at[idx_v.at[g]], buf, sem)

        def drain(g, buf, sem):
            pltpu.make_async_copy(
                tbl_hbm.at[idx_v.at[g]], buf, sem).wait()
            bat = b0 + 2 * g
            pltpu.sync_copy(buf.at[pl.ds(0, seq)], out_hbm.at[bat])
            pltpu.sync_copy(buf.at[pl.ds(seq, seq)], out_hbm.at[bat + 1])

        start(0, rows0, sem0)

        def step(g, carry):
            even = lax.rem(g, 2) == 0

            @pl.when(g + 1 < nchunk)
            def _():
                @pl.when(even)
                def _():
                    start(g + 1, rows1, sem1)

                @pl.when(jnp.logical_not(even))
                def _():
                    start(g + 1, rows0, sem0)

            @pl.when(even)
            def _():
                drain(g, rows0, sem0)

            @pl.when(jnp.logical_not(even))
            def _():
                drain(g, rows1, sem1)

            return carry

        lax.fori_loop(0, nchunk, step, 0)

    return k(x3, ptbl)


def _unpack(g3, bsz, seq):
    bb = 64

    def up(g_ref, o_ref):
        o_ref[...] = g_ref[:, :, : _D]

    return pl.pallas_call(
        up,
        grid=(bsz // bb,),
        in_specs=[pl.BlockSpec((bb, seq, 2 * _D), lambda i: (i, 0, 0))],
        out_specs=pl.BlockSpec((bb, seq, _D), lambda i: (i, 0, 0)),
        out_shape=jax.ShapeDtypeStruct((bsz, seq, _D), jnp.float32),
    )(g3)


def kernel(x, table, W, b):
    bsz, seq = x.shape
    n = bsz * seq
    xf = x.astype(jnp.int32).reshape(_NW, n // (_NW * 2 * seq), 2 * seq)
    tt = table.T
    wt = W.T
    w128 = jnp.concatenate([wt, wt], axis=1)
    b128 = jnp.concatenate([b, b]).reshape(1, 2 * _D)
    ptbl = _project_table(tt, w128, b128)
    g3 = _gather_out(xf, ptbl, bsz, seq)
    return _unpack(g3, bsz, seq)
